# baseline (speedup 1.0000x reference)
import jax
import jax.numpy as jnp
from jax import lax
from jax.experimental import pallas as pl
from jax.experimental.pallas import tpu as pltpu
from jax.experimental.pallas import tpu_sc as plsc

_B, _T, _D = 4, 8192, 1024
_ROWS = _B * _T
_NW = 32
_PPW = _T // _NW
_CHR = 8
_NCHUNK = _PPW // _CHR
_NSLOT = 3
_LANES = 16
_VPR = _D // _LANES
_XROWS = _B * _CHR


def _sc_body(x_hbm, pos_hbm, out_hbm, *refs):
    xbs = refs[0:_NSLOT]
    pbs = refs[_NSLOT : 2 * _NSLOT]
    sin = refs[2 * _NSLOT : 3 * _NSLOT]
    sout = refs[3 * _NSLOT : 4 * _NSLOT]

    c = lax.axis_index("c")
    s = lax.axis_index("s")
    wid = s * 2 + c
    pos_base = wid * _PPW

    def fetch(i, sl):
        po = pos_base + i * _CHR
        pltpu.make_async_copy(pos_hbm.at[pl.ds(po, _CHR), :], pbs[sl], sin[sl]).start()
        for b in range(_B):
            ro = b * _T + po
            pltpu.make_async_copy(
                x_hbm.at[pl.ds(ro, _CHR), :],
                xbs[sl].at[pl.ds(b * _CHR, _CHR), :],
                sin[sl],
            ).start()

    def wait_in(sl):
        pltpu.make_async_copy(pos_hbm.at[pl.ds(pos_base, _CHR), :], pbs[sl], sin[sl]).wait()
        for b in range(_B):
            pltpu.make_async_copy(
                x_hbm.at[pl.ds(pos_base, _CHR), :],
                xbs[sl].at[pl.ds(b * _CHR, _CHR), :],
                sin[sl],
            ).wait()

    def start_out(i, sl):
        po = pos_base + i * _CHR
        for b in range(_B):
            pltpu.make_async_copy(
                xbs[sl].at[pl.ds(b * _CHR, _CHR), :],
                out_hbm.at[pl.ds(b * _T + po, _CHR), :],
                sout[sl],
            ).start()

    def wait_out(sl):
        for b in range(_B):
            pltpu.make_async_copy(
                xbs[sl].at[pl.ds(b * _CHR, _CHR), :],
                out_hbm.at[pl.ds(pos_base, _CHR), :],
                sout[sl],
            ).wait()

    def compute(sl):
        xb = xbs[sl]
        pb = pbs[sl]

        @plsc.parallel_loop(0, _XROWS * _VPR, step=1, unroll=8)
        def _vbody(j):
            r = lax.shift_right_logical(j, 6)
            pr = lax.bitwise_and(r, _CHR - 1)
            col = pl.multiple_of(
                lax.shift_left(lax.bitwise_and(j, _VPR - 1), 4), _LANES
            )
            csl = pl.ds(col, _LANES)
            xb[r, csl] = xb[r, csl] + pb[pr, csl]

    def step(i, sl):
        tsl = (sl + _NSLOT - 1) % _NSLOT
        wait_in(sl)

        @pl.when(i >= 1)
        def _():
            wait_out(tsl)

        @pl.when(i + _NSLOT - 1 < _NCHUNK)
        def _():
            fetch(i + _NSLOT - 1, tsl)
        compute(sl)
        start_out(i, sl)

    for j in range(_NSLOT - 1):
        fetch(j, j)

    _MAIN = (_NCHUNK // _NSLOT) * _NSLOT

    def loop(k, _):
        for sl in range(_NSLOT):
            step(k * _NSLOT + sl, sl)
        return 0

    lax.fori_loop(0, _MAIN // _NSLOT, loop, 0)
    for i in range(_MAIN, _NCHUNK):
        step(i, i % _NSLOT)
    wait_out((_NCHUNK - 1) % _NSLOT)


def kernel(x, pos_table):
    B, T, D = x.shape
    xf = x.reshape(B * T, D)
    mesh = plsc.VectorSubcoreMesh(core_axis_name="c", subcore_axis_name="s")
    scratch = [pltpu.VMEM((_XROWS, _D), jnp.float32) for _ in range(_NSLOT)]
    scratch += [pltpu.VMEM((_CHR, _D), jnp.float32) for _ in range(_NSLOT)]
    scratch += [pltpu.SemaphoreType.DMA for _ in range(2 * _NSLOT)]
    run = pl.kernel(
        _sc_body,
        out_type=jax.ShapeDtypeStruct((_ROWS, _D), jnp.float32),
        mesh=mesh,
        scratch_types=scratch,
    )
    out = run(xf, pos_table[:T])
    return out.reshape(B, T, D)


# speedup vs baseline: 1.0164x; 1.0164x over previous
import jax
import jax.numpy as jnp
from jax import lax
from jax.experimental import pallas as pl
from jax.experimental.pallas import tpu as pltpu
from jax.experimental.pallas import tpu_sc as plsc

_B, _T, _D = 4, 8192, 1024
_ROWS = _B * _T
_NW = 32
_PPW = _T // _NW
_CHR = 8
_NCHUNK = _PPW // _CHR
_NSLOT = 2
_LANES = 16
_VPR = _D // _LANES
_XROWS = _B * _CHR


def _sc_body(x_hbm, pos_hbm, out_hbm, *refs):
    xbs = refs[0:_NSLOT]
    pbs = refs[_NSLOT : 2 * _NSLOT]
    sin = refs[2 * _NSLOT : 3 * _NSLOT]
    sout = refs[3 * _NSLOT : 4 * _NSLOT]

    c = lax.axis_index("c")
    s = lax.axis_index("s")
    wid = s * 2 + c
    pos_base = wid * _PPW

    def fetch(i, sl):
        po = pos_base + i * _CHR
        pltpu.make_async_copy(pos_hbm.at[pl.ds(po, _CHR), :], pbs[sl], sin[sl]).start()
        for b in range(_B):
            ro = b * _T + po
            pltpu.make_async_copy(
                x_hbm.at[pl.ds(ro, _CHR), :],
                xbs[sl].at[pl.ds(b * _CHR, _CHR), :],
                sin[sl],
            ).start()

    def wait_in(sl):
        pltpu.make_async_copy(pos_hbm.at[pl.ds(pos_base, _CHR), :], pbs[sl], sin[sl]).wait()
        for b in range(_B):
            pltpu.make_async_copy(
                x_hbm.at[pl.ds(pos_base, _CHR), :],
                xbs[sl].at[pl.ds(b * _CHR, _CHR), :],
                sin[sl],
            ).wait()

    def start_out(i, sl):
        po = pos_base + i * _CHR
        for b in range(_B):
            pltpu.make_async_copy(
                xbs[sl].at[pl.ds(b * _CHR, _CHR), :],
                out_hbm.at[pl.ds(b * _T + po, _CHR), :],
                sout[sl],
            ).start()

    def wait_out(sl):
        for b in range(_B):
            pltpu.make_async_copy(
                xbs[sl].at[pl.ds(b * _CHR, _CHR), :],
                out_hbm.at[pl.ds(pos_base, _CHR), :],
                sout[sl],
            ).wait()

    def compute(sl):
        xb = xbs[sl]
        pb = pbs[sl]

        @plsc.parallel_loop(0, _XROWS * _VPR, step=1, unroll=8)
        def _vbody(j):
            r = lax.shift_right_logical(j, 6)
            pr = lax.bitwise_and(r, _CHR - 1)
            col = pl.multiple_of(
                lax.shift_left(lax.bitwise_and(j, _VPR - 1), 4), _LANES
            )
            csl = pl.ds(col, _LANES)
            xb[r, csl] = xb[r, csl] + pb[pr, csl]

    def step(i, sl):
        osl = (sl + 1) % _NSLOT
        wait_in(sl)

        @pl.when(i >= 1)
        def _():
            wait_out(osl)

        @pl.when(i + 1 < _NCHUNK)
        def _():
            fetch(i + 1, osl)
        compute(sl)
        start_out(i, sl)

    fetch(0, 0)

    def loop(k, _):
        for sl in range(_NSLOT):
            step(k * _NSLOT + sl, sl)
        return 0

    lax.fori_loop(0, _NCHUNK // _NSLOT, loop, 0)
    wait_out(_NSLOT - 1)


def kernel(x, pos_table):
    B, T, D = x.shape
    xf = x.reshape(B * T, D)
    mesh = plsc.VectorSubcoreMesh(core_axis_name="c", subcore_axis_name="s")
    scratch = [pltpu.VMEM((_XROWS, _D), jnp.float32) for _ in range(_NSLOT)]
    scratch += [pltpu.VMEM((_CHR, _D), jnp.float32) for _ in range(_NSLOT)]
    scratch += [pltpu.SemaphoreType.DMA for _ in range(2 * _NSLOT)]
    run = pl.kernel(
        _sc_body,
        out_type=jax.ShapeDtypeStruct((_ROWS, _D), jnp.float32),
        mesh=mesh,
        scratch_types=scratch,
    )
    out = run(xf, pos_table[:T])
    return out.reshape(B, T, D)
